# baseline (speedup 1.0000x reference)
import functools

import jax
import jax.numpy as jnp
from jax import lax
from jax.experimental import pallas as pl
from jax.experimental.pallas import tpu as pltpu
from jax.experimental.pallas import tpu_sc as plsc

N = 10000
E = 320000
D = 128
ED = 16
H = 128

NC = 2
NS = 16
NW = NC * NS
EPW = E // NW

NSLAB = 2
EH = E // NSLAB

MB_G = 40
EPW_G = EH // NW
NIT_G = EPW_G // MB_G
NITH_G = NIT_G // 2

MB_X = 1000
SUB_X = 40
NSUB_X = MB_X // SUB_X
EPW_X = EH // NW
NIT_X = EPW_X // MB_X

EPW_S = EH // NS
MB_S = 80
SUB_S = 80
NSUB_S = MB_S // SUB_S
NIT_S = EPW_S // MB_S
NSLOT_S = 5
HN = 5120
HN2 = 5248
RPT1 = HN2 // NS
NP = 10240
RPT = NP // NS
EPW_R = EH // NW
MB_R = 200
SUB_R = 40
NSUB_R = MB_R // SUB_R
NIT_R = EPW_R // MB_R
NSLOT_R = 5

TILE_E = 4000
TILE_N = 2000

_mesh = plsc.VectorSubcoreMesh(core_axis_name="c", subcore_axis_name="s")
_mesh1 = plsc.VectorSubcoreMesh(core_axis_name="c", subcore_axis_name="s",
                                num_cores=1)


def _proj_body(h_ref, W1a_ref, W1b_ref, t1_ref, t2_ref):
    h = h_ref[...]
    t1_ref[...] = jnp.dot(h, W1a_ref[...], preferred_element_type=jnp.float32)
    t2_ref[...] = jnp.dot(h, W1b_ref[...], preferred_element_type=jnp.float32)


def _project(h, W1a, W1b):
    grid = N // TILE_N
    return pl.pallas_call(
        _proj_body,
        grid=(grid,),
        in_specs=[
            pl.BlockSpec((TILE_N, D), lambda i: (i, 0)),
            pl.BlockSpec((D, D), lambda i: (0, 0)),
            pl.BlockSpec((D, D), lambda i: (0, 0)),
        ],
        out_specs=[
            pl.BlockSpec((TILE_N, D), lambda i: (i, 0)),
            pl.BlockSpec((TILE_N, D), lambda i: (i, 0)),
        ],
        out_shape=[
            jax.ShapeDtypeStruct((N, D), jnp.float32),
            jax.ShapeDtypeStruct((N, D), jnp.float32),
        ],
    )(h, W1a, W1b)


@functools.partial(
    pl.kernel,
    out_type=jax.ShapeDtypeStruct((EH, D), jnp.float32),
    mesh=_mesh,
    scratch_types=[
        pltpu.VMEM((2, MB_G), jnp.int32),
        pltpu.VMEM((2, MB_G), jnp.int32),
        pltpu.VMEM((2, MB_G, D), jnp.float32),
        pltpu.VMEM((2, MB_G, D), jnp.float32),
        pltpu.VMEM((2, MB_G, D), jnp.float32),
        pltpu.SemaphoreType.DMA,
        pltpu.SemaphoreType.DMA,
        pltpu.SemaphoreType.DMA,
        pltpu.SemaphoreType.DMA,
        pltpu.SemaphoreType.DMA,
        pltpu.SemaphoreType.DMA,
    ],
)
def _gather_kernel(t1_hbm, t2_hbm, row_hbm, col_hbm, out_hbm,
                   idxr, idxc, g1, g2, ob,
                   sem_i0, sem_i1, sem_g0, sem_g1, sem_w0, sem_w1):
    c = lax.axis_index("c")
    s = lax.axis_index("s")
    base = (s * NC + c) * EPW_G
    sem_i = (sem_i0, sem_i1)
    sem_g = (sem_g0, sem_g1)
    sem_w = (sem_w0, sem_w1)

    def fire_idx(sl, ch):
        off = base + ch * MB_G
        pltpu.async_copy(row_hbm.at[pl.ds(off, MB_G)], idxr.at[sl], sem_i[sl])
        pltpu.async_copy(col_hbm.at[pl.ds(off, MB_G)], idxc.at[sl], sem_i[sl])

    def wait_idx(sl):
        pltpu.make_async_copy(row_hbm.at[pl.ds(0, MB_G)], idxr.at[sl], sem_i[sl]).wait()
        pltpu.make_async_copy(col_hbm.at[pl.ds(0, MB_G)], idxc.at[sl], sem_i[sl]).wait()

    def fire_g(sl):
        pltpu.async_copy(t1_hbm.at[idxr.at[sl]], g1.at[sl], sem_g[sl])
        pltpu.async_copy(t2_hbm.at[idxc.at[sl]], g2.at[sl], sem_g[sl])

    def wait_g(sl):
        pltpu.make_async_copy(t1_hbm.at[idxr.at[sl]], g1.at[sl], sem_g[sl]).wait()
        pltpu.make_async_copy(t2_hbm.at[idxc.at[sl]], g2.at[sl], sem_g[sl]).wait()

    def add(sl):
        @pl.loop(0, MB_G)
        def _r(r):
            for l in range(D // 16):
                d = pl.ds(l * 16, 16)
                ob[sl, r, d] = g1[sl, r, d] + g2[sl, r, d]

    def fire_wb(sl, ch):
        off = base + ch * MB_G
        pltpu.async_copy(ob.at[sl], out_hbm.at[pl.ds(off, MB_G)], sem_w[sl])

    def wait_wb(sl):
        pltpu.make_async_copy(ob.at[sl], out_hbm.at[pl.ds(0, MB_G)], sem_w[sl]).wait()

    fire_idx(0, 0)
    wait_idx(0)
    fire_g(0)
    fire_idx(1, 1)

    @pl.loop(0, NITH_G)
    def _i(i):
        a = 2 * i
        wait_g(0)
        wait_idx(1)
        fire_g(1)
        fire_idx(0, a + 2)

        @pl.when(i > 0)
        def _():
            wait_wb(0)

        add(0)
        fire_wb(0, a)

        wait_g(1)
        wait_idx(0)
        fire_g(0)

        @pl.when(i < NITH_G - 1)
        def _():
            fire_idx(1, a + 3)

        @pl.when(i > 0)
        def _():
            wait_wb(1)

        add(1)
        fire_wb(1, a + 1)

    wait_g(0)
    wait_wb(0)
    add(0)
    fire_wb(0, NIT_G - 1)
    wait_wb(1)
    wait_wb(0)


@functools.partial(
    pl.kernel,
    out_type=jax.ShapeDtypeStruct((EH, 16), jnp.float32),
    mesh=_mesh,
    scratch_types=[
        pltpu.VMEM((2, NSUB_X, SUB_X), jnp.int32),
        pltpu.VMEM((2, NSUB_X, SUB_X), jnp.int32),
        pltpu.VMEM((2, MB_X, 16), jnp.float32),
        pltpu.VMEM((2, MB_X, 16), jnp.float32),
        pltpu.VMEM((2, MB_X, 16), jnp.float32),
        pltpu.SemaphoreType.DMA,
        pltpu.SemaphoreType.DMA,
        pltpu.SemaphoreType.DMA,
        pltpu.SemaphoreType.DMA,
        pltpu.SemaphoreType.DMA,
        pltpu.SemaphoreType.DMA,
    ],
    compiler_params=pltpu.CompilerParams(use_tc_tiling_on_sc=False),
)
def _xgather_kernel(x16_hbm, row_hbm, col_hbm, out_hbm,
                    idxr, idxc, xr, xc, ob,
                    sem_i0, sem_i1, sem_g0, sem_g1, sem_w0, sem_w1):
    c = lax.axis_index("c")
    s = lax.axis_index("s")
    base = (s * NC + c) * EPW_X
    sem_i = (sem_i0, sem_i1)
    sem_g = (sem_g0, sem_g1)
    sem_w = (sem_w0, sem_w1)

    def fire_idx(sl, ch):
        off = base + ch * MB_X
        for j in range(NSUB_X):
            o2 = off + j * SUB_X
            pltpu.async_copy(row_hbm.at[pl.ds(o2, SUB_X)], idxr.at[sl, j], sem_i[sl])
            pltpu.async_copy(col_hbm.at[pl.ds(o2, SUB_X)], idxc.at[sl, j], sem_i[sl])

    def wait_idx(sl):
        for j in range(NSUB_X):
            pltpu.make_async_copy(
                row_hbm.at[pl.ds(0, SUB_X)], idxr.at[sl, j], sem_i[sl]).wait()
            pltpu.make_async_copy(
                col_hbm.at[pl.ds(0, SUB_X)], idxc.at[sl, j], sem_i[sl]).wait()

    def fire_g(sl):
        for j in range(NSUB_X):
            d = pl.ds(j * SUB_X, SUB_X)
            pltpu.async_copy(x16_hbm.at[idxr.at[sl, j]], xr.at[sl, d], sem_g[sl])
            pltpu.async_copy(x16_hbm.at[idxc.at[sl, j]], xc.at[sl, d], sem_g[sl])

    def wait_g(sl):
        for j in range(NSUB_X):
            d = pl.ds(j * SUB_X, SUB_X)
            pltpu.make_async_copy(
                x16_hbm.at[idxr.at[sl, j]], xr.at[sl, d], sem_g[sl]).wait()
            pltpu.make_async_copy(
                x16_hbm.at[idxc.at[sl, j]], xc.at[sl, d], sem_g[sl]).wait()

    def sub(sl):
        @pl.loop(0, MB_X)
        def _r(r):
            ob[sl, r, :] = xr[sl, r, :] - xc[sl, r, :]

    def fire_wb(sl, ch):
        off = base + ch * MB_X
        pltpu.async_copy(ob.at[sl], out_hbm.at[pl.ds(off, MB_X)], sem_w[sl])

    def wait_wb(sl):
        pltpu.make_async_copy(ob.at[sl], out_hbm.at[pl.ds(0, MB_X)], sem_w[sl]).wait()

    fire_idx(0, 0)
    wait_idx(0)
    fire_g(0)
    fire_idx(1, 1)

    @pl.loop(0, NIT_X // 2)
    def _i(i):
        a = 2 * i
        wait_g(0)
        wait_idx(1)
        fire_g(1)
        fire_idx(0, a + 2)

        @pl.when(i > 0)
        def _():
            wait_wb(0)

        sub(0)
        fire_wb(0, a)

        wait_g(1)
        wait_idx(0)
        fire_g(0)

        @pl.when(i < NIT_X // 2 - 1)
        def _():
            fire_idx(1, a + 3)

        @pl.when(i > 0)
        def _():
            wait_wb(1)

        sub(1)
        fire_wb(1, a + 1)

    wait_g(0)
    wait_wb(0)
    sub(0)
    fire_wb(0, NIT_X - 1)
    wait_wb(1)
    wait_wb(0)


def _edge_mlp_body(g_ref, relg_ref, ea_ref, Wea_ref, wsq_ref, wzd_ref, bm1_ref,
                   Wm2_ref, bm2_ref, Wc1_ref, bc1_ref, Wc2_ref, bc2_ref,
                   Wv1_ref, bv1_ref, Wv2_ref, bv2_ref, msg_ref, rel16_ref):
    gh = g_ref[...]
    rel = relg_ref[...][:, :8]
    sq = jnp.sum(rel * rel, axis=1, keepdims=True)
    zd = rel[:, 2:3]
    t1 = (gh
          + jnp.dot(ea_ref[...], Wea_ref[...], preferred_element_type=jnp.float32)
          + sq * wsq_ref[...]
          + zd * wzd_ref[...]
          + bm1_ref[...])
    h1 = jax.nn.silu(t1)
    msg = jax.nn.silu(
        jnp.dot(h1, Wm2_ref[...], preferred_element_type=jnp.float32) + bm2_ref[...])
    msg_ref[...] = msg
    hc = jax.nn.silu(
        jnp.dot(msg, Wc1_ref[...], preferred_element_type=jnp.float32) + bc1_ref[...])
    xw = (jnp.dot(hc, Wc2_ref[...], preferred_element_type=jnp.float32)
          + bc2_ref[...]) / (sq + 1e-8)
    hv = jax.nn.silu(
        jnp.dot(msg, Wv1_ref[...], preferred_element_type=jnp.float32) + bv1_ref[...])
    vw = (jnp.dot(hv, Wv2_ref[...], preferred_element_type=jnp.float32)
          + bv2_ref[...]) / (sq + 1e-8)
    rel16_ref[...] = jnp.concatenate([rel * xw, rel * vw], axis=1)


def _edge_mlp(g, relg, edge_attr, slab, Wea, wsq, wzd, bm1, Wm2, bm2,
              Wc1, bc1, Wc2, bc2, Wv1, bv1, Wv2, bv2):
    grid = EH // TILE_E
    off = slab * (EH // TILE_E)
    full = lambda i: (0, 0)
    return pl.pallas_call(
        _edge_mlp_body,
        grid=(grid,),
        in_specs=[
            pl.BlockSpec((TILE_E, D), lambda i: (i, 0)),
            pl.BlockSpec((TILE_E, 16), lambda i: (i, 0)),
            pl.BlockSpec((TILE_E, ED), lambda i: (i + off, 0)),
            pl.BlockSpec((ED, H), full),
            pl.BlockSpec((1, H), full),
            pl.BlockSpec((1, H), full),
            pl.BlockSpec((1, H), full),
            pl.BlockSpec((H, H), full),
            pl.BlockSpec((1, H), full),
            pl.BlockSpec((H, H), full),
            pl.BlockSpec((1, H), full),
            pl.BlockSpec((H, 1), full),
            pl.BlockSpec((1, 1), full),
            pl.BlockSpec((H, H), full),
            pl.BlockSpec((1, H), full),
            pl.BlockSpec((H, 1), full),
            pl.BlockSpec((1, 1), full),
        ],
        out_specs=[
            pl.BlockSpec((TILE_E, H), lambda i: (i, 0)),
            pl.BlockSpec((TILE_E, 16), lambda i: (i, 0)),
        ],
        out_shape=[
            jax.ShapeDtypeStruct((EH, H), jnp.float32),
            jax.ShapeDtypeStruct((EH, 16), jnp.float32),
        ],
    )(g, relg, edge_attr, Wea, wsq, wzd, bm1, Wm2, bm2,
      Wc1, bc1, Wc2, bc2, Wv1, bv1, Wv2, bv2)


@functools.partial(
    pl.kernel,
    out_type=jax.ShapeDtypeStruct((NC, HN2, H), jnp.float32),
    mesh=_mesh,
    scratch_types=[
        pltpu.VMEM((NSLOT_S, NSUB_S, SUB_S), jnp.int32),
        pltpu.VMEM((NSLOT_S, MB_S, H), jnp.float32),
        pltpu.VMEM_SHARED((HN2, H), jnp.float32),
        pltpu.SemaphoreType.DMA,
        pltpu.SemaphoreType.DMA,
        pltpu.SemaphoreType.DMA,
        pltpu.SemaphoreType.DMA,
        pltpu.SemaphoreType.DMA,
        pltpu.SemaphoreType.DMA,
    ],
)
def _scatter_msg_kernel(msg_hbm, row_hbm, zmsg_hbm, pmsg_hbm,
                        idxb, msgb, smsg,
                        sem_l0, sem_l1, sem_l2, sem_l3, sem_l4, sem_z):
    c = lax.axis_index("c")
    s = lax.axis_index("s")
    base = s * EPW_S
    r0 = s * RPT1
    cbase = c * HN
    trash = HN + s
    sem_l = (sem_l0, sem_l1, sem_l2, sem_l3, sem_l4)

    pltpu.async_copy(zmsg_hbm.at[pl.ds(r0, RPT1)],
                     smsg.at[pl.ds(r0, RPT1)], sem_z).wait()
    plsc.subcore_barrier()

    def fire_loads(sl, ch):
        off = base + ch * MB_S
        for j in range(NSUB_S):
            pltpu.async_copy(
                row_hbm.at[pl.ds(off + j * SUB_S, SUB_S)], idxb.at[sl, j], sem_l[sl])
        pltpu.async_copy(msg_hbm.at[pl.ds(off, MB_S)], msgb.at[sl], sem_l[sl])

    def wait_loads(sl):
        for j in range(NSUB_S):
            pltpu.make_async_copy(
                row_hbm.at[pl.ds(0, SUB_S)], idxb.at[sl, j], sem_l[sl]).wait()
        pltpu.make_async_copy(msg_hbm.at[pl.ds(0, MB_S)], msgb.at[sl], sem_l[sl]).wait()

    def turn(sl, ch):
        wait_loads(sl)
        for j in range(NSUB_S):
            for k in range(SUB_S // 16):
                d = pl.ds(k * 16, 16)
                vv = idxb[sl, j, d] - cbase
                ok = (vv >= 0) & (vv < HN)
                idxb[sl, j, d] = jnp.where(ok, vv, trash)
        for j in range(NSUB_S):
            pltpu.sync_copy(msgb.at[sl, pl.ds(j * SUB_S, SUB_S)],
                            smsg.at[idxb.at[sl, j]], add=True)

        @pl.when(ch + 3 < NIT_S)
        def _():
            fire_loads((sl + 3) % NSLOT_S, ch + 3)

    for p in range(3):
        fire_loads(p, p)

    @pl.loop(0, NIT_S // NSLOT_S)
    def _i(i):
        cc = i * NSLOT_S
        for p in range(NSLOT_S):
            turn(p, cc + p)

    plsc.subcore_barrier()
    pltpu.async_copy(smsg.at[pl.ds(r0, RPT1)],
                     pmsg_hbm.at[c, pl.ds(r0, RPT1)], sem_z).wait()


@functools.partial(
    pl.kernel,
    out_type=jax.ShapeDtypeStruct((NC, NP, 16), jnp.float32),
    mesh=_mesh,
    scratch_types=[
        pltpu.VMEM((NSLOT_R, NSUB_R, SUB_R), jnp.int32),
        pltpu.VMEM((NSLOT_R, MB_R, 16), jnp.float32),
        pltpu.VMEM_SHARED((NP, 16), jnp.float32),
        pltpu.SemaphoreType.DMA,
        pltpu.SemaphoreType.DMA,
        pltpu.SemaphoreType.DMA,
        pltpu.SemaphoreType.DMA,
        pltpu.SemaphoreType.DMA,
        pltpu.SemaphoreType.DMA,
    ],
    compiler_params=pltpu.CompilerParams(use_tc_tiling_on_sc=False),
)
def _scatter_rel_kernel(rel_hbm, row_hbm, zrel_hbm, prel_hbm,
                        idxb, relb, srel,
                        sem_l0, sem_l1, sem_l2, sem_l3, sem_l4, sem_z):
    c = lax.axis_index("c")
    s = lax.axis_index("s")
    base = (s * NC + c) * EPW_R
    r0 = s * RPT
    sem_l = (sem_l0, sem_l1, sem_l2, sem_l3, sem_l4)

    pltpu.async_copy(zrel_hbm.at[pl.ds(r0, RPT)],
                     srel.at[pl.ds(r0, RPT)], sem_z).wait()
    plsc.subcore_barrier()

    def fire_loads(sl, ch):
        off = base + ch * MB_R
        for j in range(NSUB_R):
            pltpu.async_copy(
                row_hbm.at[pl.ds(off + j * SUB_R, SUB_R)], idxb.at[sl, j], sem_l[sl])
        pltpu.async_copy(rel_hbm.at[pl.ds(off, MB_R)], relb.at[sl], sem_l[sl])

    def wait_loads(sl):
        for j in range(NSUB_R):
            pltpu.make_async_copy(
                row_hbm.at[pl.ds(0, SUB_R)], idxb.at[sl, j], sem_l[sl]).wait()
        pltpu.make_async_copy(rel_hbm.at[pl.ds(0, MB_R)], relb.at[sl], sem_l[sl]).wait()

    def turn(sl, ch):
        wait_loads(sl)
        for j in range(NSUB_R):
            pltpu.sync_copy(relb.at[sl, pl.ds(j * SUB_R, SUB_R)],
                            srel.at[idxb.at[sl, j]], add=True)

        @pl.when(ch + 3 < NIT_R)
        def _():
            fire_loads((sl + 3) % NSLOT_R, ch + 3)

    for p in range(3):
        fire_loads(p, p)

    @pl.loop(0, NIT_R // NSLOT_R)
    def _i(i):
        cc = i * NSLOT_R
        for p in range(NSLOT_R):
            turn(p, cc + p)

    plsc.subcore_barrier()
    pltpu.async_copy(srel.at[pl.ds(r0, RPT)],
                     prel_hbm.at[c, pl.ds(r0, RPT)], sem_z).wait()


def _node_body(h_ref, pm_ref, pr_ref, Wn1a_ref, Wn1b_ref, bn1_ref,
               Wn2_ref, bn2_ref, hnew_ref, xv_ref):
    h = h_ref[...]
    aggr = pm_ref[...]
    t = jax.nn.silu(
        jnp.dot(h, Wn1a_ref[...], preferred_element_type=jnp.float32)
        + jnp.dot(aggr, Wn1b_ref[...], preferred_element_type=jnp.float32)
        + bn1_ref[...])
    hnew_ref[...] = h + jnp.dot(t, Wn2_ref[...],
                                preferred_element_type=jnp.float32) + bn2_ref[...]
    xv_ref[...] = pr_ref[...]


def _node_update(h, pmsg, prel, Wn1a, Wn1b, bn1, Wn2, bn2):
    grid = N // TILE_N
    full = lambda i: (0, 0)
    return pl.pallas_call(
        _node_body,
        grid=(grid,),
        in_specs=[
            pl.BlockSpec((TILE_N, D), lambda i: (i, 0)),
            pl.BlockSpec((TILE_N, H), lambda i: (i, 0)),
            pl.BlockSpec((TILE_N, 16), lambda i: (i, 0)),
            pl.BlockSpec((D, H), full),
            pl.BlockSpec((H, H), full),
            pl.BlockSpec((1, H), full),
            pl.BlockSpec((H, D), full),
            pl.BlockSpec((1, D), full),
        ],
        out_specs=[
            pl.BlockSpec((TILE_N, D), lambda i: (i, 0)),
            pl.BlockSpec((TILE_N, 16), lambda i: (i, 0)),
        ],
        out_shape=[
            jax.ShapeDtypeStruct((N, D), jnp.float32),
            jax.ShapeDtypeStruct((N, 16), jnp.float32),
        ],
    )(h, pmsg, prel, Wn1a, Wn1b, bn1, Wn2, bn2)


def kernel(h, x, v, edge_index, edge_attr,
           Wm1, bm1, Wm2, bm2,
           Wn1, bn1, Wn2, bn2,
           Wc1, bc1, Wc2, bc2,
           Wv1, bv1, Wv2, bv2):
    row = edge_index[0]
    col = edge_index[1]
    x16 = jnp.pad(x, ((0, 0), (0, 13)))

    W1a = Wm1[:D]
    W1b = Wm1[D:2 * D]
    wsq = Wm1[2 * D:2 * D + 1]
    wzd = Wm1[2 * D + 1:2 * D + 2]
    Wea = Wm1[2 * D + 2:]

    t1, t2 = _project(h, W1a, W1b)
    zmsg = jnp.zeros((HN2, H), jnp.float32)
    zrel = jnp.zeros((NP, 16), jnp.float32)

    pmsgs, prels = [], []
    for k in range(NSLAB):
        sl = slice(k * EH, (k + 1) * EH)
        g = _gather_kernel(t1, t2, row[sl], col[sl])
        relg = _xgather_kernel(x16, row[sl], col[sl])
        msg, rel16 = _edge_mlp(
            g, relg, edge_attr, k, Wea, wsq, wzd, bm1.reshape(1, H), Wm2,
            bm2.reshape(1, H), Wc1, bc1.reshape(1, H), Wc2, bc2.reshape(1, 1),
            Wv1, bv1.reshape(1, H), Wv2, bv2.reshape(1, 1))
        pmsgs.append(_scatter_msg_kernel(msg, row[sl], zmsg))
        prels.append(_scatter_rel_kernel(rel16, row[sl], zrel))

    pmsg = pmsgs[0] + pmsgs[1]
    prel = prels[0][0] + prels[0][1] + prels[1][0] + prels[1][1]
    aggr = jnp.concatenate([pmsg[0, :HN], pmsg[1, :N - HN]], axis=0)

    h_new, xv = _node_update(
        h, aggr, prel, Wn1[:D], Wn1[D:], bn1.reshape(1, H), Wn2,
        bn2.reshape(1, D))
    x_new = x + xv[:, 0:3]
    v_new = v + xv[:, 8:11]
    return (h_new, x_new, v_new)


# speedup vs baseline: 1.0082x; 1.0082x over previous
import functools

import jax
import jax.numpy as jnp
from jax import lax
from jax.experimental import pallas as pl
from jax.experimental.pallas import tpu as pltpu
from jax.experimental.pallas import tpu_sc as plsc

N = 10000
E = 320000
D = 128
ED = 16
H = 128

NC = 2
NS = 16
NW = NC * NS
EPW = E // NW

NSLAB = 2
EH = E // NSLAB

MB_G = 40
EPW_G = EH // NW
NIT_G = EPW_G // MB_G
NITH_G = NIT_G // 2

MB_X = 1000
SUB_X = 40
NSUB_X = MB_X // SUB_X
EPW_X = EH // NW
NIT_X = EPW_X // MB_X

EPW_S = EH // NS
MB_S = 80
SUB_S = 80
NSUB_S = MB_S // SUB_S
NIT_S = EPW_S // MB_S
NSLOT_S = 5
HN = 5120
HN2 = 5248
RPT1 = HN2 // NS
NP = 10240
RPT = NP // NS
EPW_R = EH // NS
MB_R = 400
SUB_R = 80
NSUB_R = MB_R // SUB_R
NIT_R = EPW_R // MB_R
NSLOT_R = 5

TILE_E = 4000
TILE_N = 2000

_mesh = plsc.VectorSubcoreMesh(core_axis_name="c", subcore_axis_name="s")
_mesh1 = plsc.VectorSubcoreMesh(core_axis_name="c", subcore_axis_name="s",
                                num_cores=1)


def _proj_body(h_ref, W1a_ref, W1b_ref, t1_ref, t2_ref):
    h = h_ref[...]
    t1_ref[...] = jnp.dot(h, W1a_ref[...], preferred_element_type=jnp.float32)
    t2_ref[...] = jnp.dot(h, W1b_ref[...], preferred_element_type=jnp.float32)


def _project(h, W1a, W1b):
    grid = N // TILE_N
    return pl.pallas_call(
        _proj_body,
        grid=(grid,),
        in_specs=[
            pl.BlockSpec((TILE_N, D), lambda i: (i, 0)),
            pl.BlockSpec((D, D), lambda i: (0, 0)),
            pl.BlockSpec((D, D), lambda i: (0, 0)),
        ],
        out_specs=[
            pl.BlockSpec((TILE_N, D), lambda i: (i, 0)),
            pl.BlockSpec((TILE_N, D), lambda i: (i, 0)),
        ],
        out_shape=[
            jax.ShapeDtypeStruct((N, D), jnp.float32),
            jax.ShapeDtypeStruct((N, D), jnp.float32),
        ],
    )(h, W1a, W1b)


@functools.partial(
    pl.kernel,
    out_type=jax.ShapeDtypeStruct((EH, D), jnp.float32),
    mesh=_mesh,
    scratch_types=[
        pltpu.VMEM((2, MB_G), jnp.int32),
        pltpu.VMEM((2, MB_G), jnp.int32),
        pltpu.VMEM((2, MB_G, D), jnp.float32),
        pltpu.VMEM((2, MB_G, D), jnp.float32),
        pltpu.VMEM((2, MB_G, D), jnp.float32),
        pltpu.SemaphoreType.DMA,
        pltpu.SemaphoreType.DMA,
        pltpu.SemaphoreType.DMA,
        pltpu.SemaphoreType.DMA,
        pltpu.SemaphoreType.DMA,
        pltpu.SemaphoreType.DMA,
    ],
)
def _gather_kernel(t1_hbm, t2_hbm, row_hbm, col_hbm, out_hbm,
                   idxr, idxc, g1, g2, ob,
                   sem_i0, sem_i1, sem_g0, sem_g1, sem_w0, sem_w1):
    c = lax.axis_index("c")
    s = lax.axis_index("s")
    base = (s * NC + c) * EPW_G
    sem_i = (sem_i0, sem_i1)
    sem_g = (sem_g0, sem_g1)
    sem_w = (sem_w0, sem_w1)

    def fire_idx(sl, ch):
        off = base + ch * MB_G
        pltpu.async_copy(row_hbm.at[pl.ds(off, MB_G)], idxr.at[sl], sem_i[sl])
        pltpu.async_copy(col_hbm.at[pl.ds(off, MB_G)], idxc.at[sl], sem_i[sl])

    def wait_idx(sl):
        pltpu.make_async_copy(row_hbm.at[pl.ds(0, MB_G)], idxr.at[sl], sem_i[sl]).wait()
        pltpu.make_async_copy(col_hbm.at[pl.ds(0, MB_G)], idxc.at[sl], sem_i[sl]).wait()

    def fire_g(sl):
        pltpu.async_copy(t1_hbm.at[idxr.at[sl]], g1.at[sl], sem_g[sl])
        pltpu.async_copy(t2_hbm.at[idxc.at[sl]], g2.at[sl], sem_g[sl])

    def wait_g(sl):
        pltpu.make_async_copy(t1_hbm.at[idxr.at[sl]], g1.at[sl], sem_g[sl]).wait()
        pltpu.make_async_copy(t2_hbm.at[idxc.at[sl]], g2.at[sl], sem_g[sl]).wait()

    def add(sl):
        @pl.loop(0, MB_G)
        def _r(r):
            for l in range(D // 16):
                d = pl.ds(l * 16, 16)
                ob[sl, r, d] = g1[sl, r, d] + g2[sl, r, d]

    def fire_wb(sl, ch):
        off = base + ch * MB_G
        pltpu.async_copy(ob.at[sl], out_hbm.at[pl.ds(off, MB_G)], sem_w[sl])

    def wait_wb(sl):
        pltpu.make_async_copy(ob.at[sl], out_hbm.at[pl.ds(0, MB_G)], sem_w[sl]).wait()

    fire_idx(0, 0)
    wait_idx(0)
    fire_g(0)
    fire_idx(1, 1)

    @pl.loop(0, NITH_G)
    def _i(i):
        a = 2 * i
        wait_g(0)
        wait_idx(1)
        fire_g(1)
        fire_idx(0, a + 2)

        @pl.when(i > 0)
        def _():
            wait_wb(0)

        add(0)
        fire_wb(0, a)

        wait_g(1)
        wait_idx(0)
        fire_g(0)

        @pl.when(i < NITH_G - 1)
        def _():
            fire_idx(1, a + 3)

        @pl.when(i > 0)
        def _():
            wait_wb(1)

        add(1)
        fire_wb(1, a + 1)

    wait_g(0)
    wait_wb(0)
    add(0)
    fire_wb(0, NIT_G - 1)
    wait_wb(1)
    wait_wb(0)


@functools.partial(
    pl.kernel,
    out_type=jax.ShapeDtypeStruct((EH, 16), jnp.float32),
    mesh=_mesh,
    scratch_types=[
        pltpu.VMEM((2, NSUB_X, SUB_X), jnp.int32),
        pltpu.VMEM((2, NSUB_X, SUB_X), jnp.int32),
        pltpu.VMEM((2, MB_X, 16), jnp.float32),
        pltpu.VMEM((2, MB_X, 16), jnp.float32),
        pltpu.VMEM((2, MB_X, 16), jnp.float32),
        pltpu.SemaphoreType.DMA,
        pltpu.SemaphoreType.DMA,
        pltpu.SemaphoreType.DMA,
        pltpu.SemaphoreType.DMA,
        pltpu.SemaphoreType.DMA,
        pltpu.SemaphoreType.DMA,
    ],
    compiler_params=pltpu.CompilerParams(use_tc_tiling_on_sc=False),
)
def _xgather_kernel(x16_hbm, row_hbm, col_hbm, out_hbm,
                    idxr, idxc, xr, xc, ob,
                    sem_i0, sem_i1, sem_g0, sem_g1, sem_w0, sem_w1):
    c = lax.axis_index("c")
    s = lax.axis_index("s")
    base = (s * NC + c) * EPW_X
    sem_i = (sem_i0, sem_i1)
    sem_g = (sem_g0, sem_g1)
    sem_w = (sem_w0, sem_w1)

    def fire_idx(sl, ch):
        off = base + ch * MB_X
        for j in range(NSUB_X):
            o2 = off + j * SUB_X
            pltpu.async_copy(row_hbm.at[pl.ds(o2, SUB_X)], idxr.at[sl, j], sem_i[sl])
            pltpu.async_copy(col_hbm.at[pl.ds(o2, SUB_X)], idxc.at[sl, j], sem_i[sl])

    def wait_idx(sl):
        for j in range(NSUB_X):
            pltpu.make_async_copy(
                row_hbm.at[pl.ds(0, SUB_X)], idxr.at[sl, j], sem_i[sl]).wait()
            pltpu.make_async_copy(
                col_hbm.at[pl.ds(0, SUB_X)], idxc.at[sl, j], sem_i[sl]).wait()

    def fire_g(sl):
        for j in range(NSUB_X):
            d = pl.ds(j * SUB_X, SUB_X)
            pltpu.async_copy(x16_hbm.at[idxr.at[sl, j]], xr.at[sl, d], sem_g[sl])
            pltpu.async_copy(x16_hbm.at[idxc.at[sl, j]], xc.at[sl, d], sem_g[sl])

    def wait_g(sl):
        for j in range(NSUB_X):
            d = pl.ds(j * SUB_X, SUB_X)
            pltpu.make_async_copy(
                x16_hbm.at[idxr.at[sl, j]], xr.at[sl, d], sem_g[sl]).wait()
            pltpu.make_async_copy(
                x16_hbm.at[idxc.at[sl, j]], xc.at[sl, d], sem_g[sl]).wait()

    def sub(sl):
        @pl.loop(0, MB_X)
        def _r(r):
            ob[sl, r, :] = xr[sl, r, :] - xc[sl, r, :]

    def fire_wb(sl, ch):
        off = base + ch * MB_X
        pltpu.async_copy(ob.at[sl], out_hbm.at[pl.ds(off, MB_X)], sem_w[sl])

    def wait_wb(sl):
        pltpu.make_async_copy(ob.at[sl], out_hbm.at[pl.ds(0, MB_X)], sem_w[sl]).wait()

    fire_idx(0, 0)
    wait_idx(0)
    fire_g(0)
    fire_idx(1, 1)

    @pl.loop(0, NIT_X // 2)
    def _i(i):
        a = 2 * i
        wait_g(0)
        wait_idx(1)
        fire_g(1)
        fire_idx(0, a + 2)

        @pl.when(i > 0)
        def _():
            wait_wb(0)

        sub(0)
        fire_wb(0, a)

        wait_g(1)
        wait_idx(0)
        fire_g(0)

        @pl.when(i < NIT_X // 2 - 1)
        def _():
            fire_idx(1, a + 3)

        @pl.when(i > 0)
        def _():
            wait_wb(1)

        sub(1)
        fire_wb(1, a + 1)

    wait_g(0)
    wait_wb(0)
    sub(0)
    fire_wb(0, NIT_X - 1)
    wait_wb(1)
    wait_wb(0)


def _edge_mlp_body(g_ref, relg_ref, ea_ref, Wea_ref, wsq_ref, wzd_ref, bm1_ref,
                   Wm2_ref, bm2_ref, Wc1_ref, bc1_ref, Wc2_ref, bc2_ref,
                   Wv1_ref, bv1_ref, Wv2_ref, bv2_ref, msg_ref, rel16_ref):
    gh = g_ref[...]
    rel = relg_ref[...][:, :8]
    sq = jnp.sum(rel * rel, axis=1, keepdims=True)
    zd = rel[:, 2:3]
    t1 = (gh
          + jnp.dot(ea_ref[...], Wea_ref[...], preferred_element_type=jnp.float32)
          + sq * wsq_ref[...]
          + zd * wzd_ref[...]
          + bm1_ref[...])
    h1 = jax.nn.silu(t1)
    msg = jax.nn.silu(
        jnp.dot(h1, Wm2_ref[...], preferred_element_type=jnp.float32) + bm2_ref[...])
    msg_ref[...] = msg
    hc = jax.nn.silu(
        jnp.dot(msg, Wc1_ref[...], preferred_element_type=jnp.float32) + bc1_ref[...])
    xw = (jnp.dot(hc, Wc2_ref[...], preferred_element_type=jnp.float32)
          + bc2_ref[...]) / (sq + 1e-8)
    hv = jax.nn.silu(
        jnp.dot(msg, Wv1_ref[...], preferred_element_type=jnp.float32) + bv1_ref[...])
    vw = (jnp.dot(hv, Wv2_ref[...], preferred_element_type=jnp.float32)
          + bv2_ref[...]) / (sq + 1e-8)
    rel16_ref[...] = jnp.concatenate([rel * xw, rel * vw], axis=1)


def _edge_mlp(g, relg, edge_attr, slab, Wea, wsq, wzd, bm1, Wm2, bm2,
              Wc1, bc1, Wc2, bc2, Wv1, bv1, Wv2, bv2):
    grid = EH // TILE_E
    off = slab * (EH // TILE_E)
    full = lambda i: (0, 0)
    return pl.pallas_call(
        _edge_mlp_body,
        grid=(grid,),
        in_specs=[
            pl.BlockSpec((TILE_E, D), lambda i: (i, 0)),
            pl.BlockSpec((TILE_E, 16), lambda i: (i, 0)),
            pl.BlockSpec((TILE_E, ED), lambda i: (i + off, 0)),
            pl.BlockSpec((ED, H), full),
            pl.BlockSpec((1, H), full),
            pl.BlockSpec((1, H), full),
            pl.BlockSpec((1, H), full),
            pl.BlockSpec((H, H), full),
            pl.BlockSpec((1, H), full),
            pl.BlockSpec((H, H), full),
            pl.BlockSpec((1, H), full),
            pl.BlockSpec((H, 1), full),
            pl.BlockSpec((1, 1), full),
            pl.BlockSpec((H, H), full),
            pl.BlockSpec((1, H), full),
            pl.BlockSpec((H, 1), full),
            pl.BlockSpec((1, 1), full),
        ],
        out_specs=[
            pl.BlockSpec((TILE_E, H), lambda i: (i, 0)),
            pl.BlockSpec((TILE_E, 16), lambda i: (i, 0)),
        ],
        out_shape=[
            jax.ShapeDtypeStruct((EH, H), jnp.float32),
            jax.ShapeDtypeStruct((EH, 16), jnp.float32),
        ],
    )(g, relg, edge_attr, Wea, wsq, wzd, bm1, Wm2, bm2,
      Wc1, bc1, Wc2, bc2, Wv1, bv1, Wv2, bv2)


@functools.partial(
    pl.kernel,
    out_type=jax.ShapeDtypeStruct((NC, HN2, H), jnp.float32),
    mesh=_mesh,
    scratch_types=[
        pltpu.VMEM((NSLOT_S, NSUB_S, SUB_S), jnp.int32),
        pltpu.VMEM((NSLOT_S, MB_S, H), jnp.float32),
        pltpu.VMEM_SHARED((HN2, H), jnp.float32),
        pltpu.SemaphoreType.DMA,
        pltpu.SemaphoreType.DMA,
        pltpu.SemaphoreType.DMA,
        pltpu.SemaphoreType.DMA,
        pltpu.SemaphoreType.DMA,
        pltpu.SemaphoreType.DMA,
    ],
)
def _scatter_msg_kernel(msg_hbm, row_hbm, zmsg_hbm, pmsg_hbm,
                        idxb, msgb, smsg,
                        sem_l0, sem_l1, sem_l2, sem_l3, sem_l4, sem_z):
    c = lax.axis_index("c")
    s = lax.axis_index("s")
    base = s * EPW_S
    r0 = s * RPT1
    cbase = c * HN
    trash = HN + s
    sem_l = (sem_l0, sem_l1, sem_l2, sem_l3, sem_l4)

    pltpu.async_copy(zmsg_hbm.at[pl.ds(r0, RPT1)],
                     smsg.at[pl.ds(r0, RPT1)], sem_z).wait()
    plsc.subcore_barrier()

    def fire_loads(sl, ch):
        off = base + ch * MB_S
        for j in range(NSUB_S):
            pltpu.async_copy(
                row_hbm.at[pl.ds(off + j * SUB_S, SUB_S)], idxb.at[sl, j], sem_l[sl])
        pltpu.async_copy(msg_hbm.at[pl.ds(off, MB_S)], msgb.at[sl], sem_l[sl])

    def wait_loads(sl):
        for j in range(NSUB_S):
            pltpu.make_async_copy(
                row_hbm.at[pl.ds(0, SUB_S)], idxb.at[sl, j], sem_l[sl]).wait()
        pltpu.make_async_copy(msg_hbm.at[pl.ds(0, MB_S)], msgb.at[sl], sem_l[sl]).wait()

    def turn(sl, ch):
        wait_loads(sl)
        for j in range(NSUB_S):
            for k in range(SUB_S // 16):
                d = pl.ds(k * 16, 16)
                vv = idxb[sl, j, d] - cbase
                ok = (vv >= 0) & (vv < HN)
                idxb[sl, j, d] = jnp.where(ok, vv, trash)
        for j in range(NSUB_S):
            pltpu.sync_copy(msgb.at[sl, pl.ds(j * SUB_S, SUB_S)],
                            smsg.at[idxb.at[sl, j]], add=True)

        @pl.when(ch + 3 < NIT_S)
        def _():
            fire_loads((sl + 3) % NSLOT_S, ch + 3)

    for p in range(3):
        fire_loads(p, p)

    @pl.loop(0, NIT_S // NSLOT_S)
    def _i(i):
        cc = i * NSLOT_S
        for p in range(NSLOT_S):
            turn(p, cc + p)

    plsc.subcore_barrier()
    pltpu.async_copy(smsg.at[pl.ds(r0, RPT1)],
                     pmsg_hbm.at[c, pl.ds(r0, RPT1)], sem_z).wait()


@functools.partial(
    pl.kernel,
    out_type=jax.ShapeDtypeStruct((NP, 16), jnp.float32),
    mesh=_mesh1,
    scratch_types=[
        pltpu.VMEM((NSLOT_R, NSUB_R, SUB_R), jnp.int32),
        pltpu.VMEM((NSLOT_R, MB_R, 16), jnp.float32),
        pltpu.VMEM_SHARED((NP, 16), jnp.float32),
        pltpu.SemaphoreType.DMA,
        pltpu.SemaphoreType.DMA,
        pltpu.SemaphoreType.DMA,
        pltpu.SemaphoreType.DMA,
        pltpu.SemaphoreType.DMA,
        pltpu.SemaphoreType.DMA,
    ],
    compiler_params=pltpu.CompilerParams(use_tc_tiling_on_sc=False),
)
def _scatter_rel_kernel(rel_hbm, row_hbm, zrel_hbm, prel_hbm,
                        idxb, relb, srel,
                        sem_l0, sem_l1, sem_l2, sem_l3, sem_l4, sem_z):
    s = lax.axis_index("s")
    base = s * EPW_R
    r0 = s * RPT
    sem_l = (sem_l0, sem_l1, sem_l2, sem_l3, sem_l4)

    pltpu.async_copy(zrel_hbm.at[pl.ds(r0, RPT)],
                     srel.at[pl.ds(r0, RPT)], sem_z).wait()
    plsc.subcore_barrier()

    def fire_loads(sl, ch):
        off = base + ch * MB_R
        for j in range(NSUB_R):
            pltpu.async_copy(
                row_hbm.at[pl.ds(off + j * SUB_R, SUB_R)], idxb.at[sl, j], sem_l[sl])
        pltpu.async_copy(rel_hbm.at[pl.ds(off, MB_R)], relb.at[sl], sem_l[sl])

    def wait_loads(sl):
        for j in range(NSUB_R):
            pltpu.make_async_copy(
                row_hbm.at[pl.ds(0, SUB_R)], idxb.at[sl, j], sem_l[sl]).wait()
        pltpu.make_async_copy(rel_hbm.at[pl.ds(0, MB_R)], relb.at[sl], sem_l[sl]).wait()

    def turn(sl, ch):
        wait_loads(sl)
        for j in range(NSUB_R):
            pltpu.sync_copy(relb.at[sl, pl.ds(j * SUB_R, SUB_R)],
                            srel.at[idxb.at[sl, j]], add=True)

        @pl.when(ch + 3 < NIT_R)
        def _():
            fire_loads((sl + 3) % NSLOT_R, ch + 3)

    for p in range(3):
        fire_loads(p, p)

    @pl.loop(0, NIT_R // NSLOT_R)
    def _i(i):
        cc = i * NSLOT_R
        for p in range(NSLOT_R):
            turn(p, cc + p)

    plsc.subcore_barrier()
    pltpu.async_copy(srel.at[pl.ds(r0, RPT)],
                     prel_hbm.at[pl.ds(r0, RPT)], sem_z).wait()


def _node_body(h_ref, pm_ref, pr_ref, Wn1a_ref, Wn1b_ref, bn1_ref,
               Wn2_ref, bn2_ref, hnew_ref, xv_ref):
    h = h_ref[...]
    aggr = pm_ref[...]
    t = jax.nn.silu(
        jnp.dot(h, Wn1a_ref[...], preferred_element_type=jnp.float32)
        + jnp.dot(aggr, Wn1b_ref[...], preferred_element_type=jnp.float32)
        + bn1_ref[...])
    hnew_ref[...] = h + jnp.dot(t, Wn2_ref[...],
                                preferred_element_type=jnp.float32) + bn2_ref[...]
    xv_ref[...] = pr_ref[...]


def _node_update(h, pmsg, prel, Wn1a, Wn1b, bn1, Wn2, bn2):
    grid = N // TILE_N
    full = lambda i: (0, 0)
    return pl.pallas_call(
        _node_body,
        grid=(grid,),
        in_specs=[
            pl.BlockSpec((TILE_N, D), lambda i: (i, 0)),
            pl.BlockSpec((TILE_N, H), lambda i: (i, 0)),
            pl.BlockSpec((TILE_N, 16), lambda i: (i, 0)),
            pl.BlockSpec((D, H), full),
            pl.BlockSpec((H, H), full),
            pl.BlockSpec((1, H), full),
            pl.BlockSpec((H, D), full),
            pl.BlockSpec((1, D), full),
        ],
        out_specs=[
            pl.BlockSpec((TILE_N, D), lambda i: (i, 0)),
            pl.BlockSpec((TILE_N, 16), lambda i: (i, 0)),
        ],
        out_shape=[
            jax.ShapeDtypeStruct((N, D), jnp.float32),
            jax.ShapeDtypeStruct((N, 16), jnp.float32),
        ],
    )(h, pmsg, prel, Wn1a, Wn1b, bn1, Wn2, bn2)


def kernel(h, x, v, edge_index, edge_attr,
           Wm1, bm1, Wm2, bm2,
           Wn1, bn1, Wn2, bn2,
           Wc1, bc1, Wc2, bc2,
           Wv1, bv1, Wv2, bv2):
    row = edge_index[0]
    col = edge_index[1]
    x16 = jnp.pad(x, ((0, 0), (0, 13)))

    W1a = Wm1[:D]
    W1b = Wm1[D:2 * D]
    wsq = Wm1[2 * D:2 * D + 1]
    wzd = Wm1[2 * D + 1:2 * D + 2]
    Wea = Wm1[2 * D + 2:]

    t1, t2 = _project(h, W1a, W1b)
    zmsg = jnp.zeros((HN2, H), jnp.float32)
    zrel = jnp.zeros((NP, 16), jnp.float32)

    pmsgs, prels = [], []
    for k in range(NSLAB):
        sl = slice(k * EH, (k + 1) * EH)
        g = _gather_kernel(t1, t2, row[sl], col[sl])
        relg = _xgather_kernel(x16, row[sl], col[sl])
        msg, rel16 = _edge_mlp(
            g, relg, edge_attr, k, Wea, wsq, wzd, bm1.reshape(1, H), Wm2,
            bm2.reshape(1, H), Wc1, bc1.reshape(1, H), Wc2, bc2.reshape(1, 1),
            Wv1, bv1.reshape(1, H), Wv2, bv2.reshape(1, 1))
        pmsgs.append(_scatter_msg_kernel(msg, row[sl], zmsg))
        prels.append(_scatter_rel_kernel(rel16, row[sl], zrel))

    pmsg = pmsgs[0] + pmsgs[1]
    prel = prels[0] + prels[1]
    aggr = jnp.concatenate([pmsg[0, :HN], pmsg[1, :N - HN]], axis=0)

    h_new, xv = _node_update(
        h, aggr, prel, Wn1[:D], Wn1[D:], bn1.reshape(1, H), Wn2,
        bn2.reshape(1, D))
    x_new = x + xv[:, 0:3]
    v_new = v + xv[:, 8:11]
    return (h_new, x_new, v_new)
